# Initial kernel scaffold; baseline (speedup 1.0000x reference)
import jax
import jax.numpy as jnp
from jax.experimental import pallas as pl


def kernel(slots, inputs, Wq, Wk, Wv, Wu, Wg):
    raise NotImplementedError("write your pallas kernel here")


# speedup vs baseline: 1.2258x; 1.2258x over previous
import functools

import jax
import jax.numpy as jnp
from jax.experimental import pallas as pl

_LN_EPS = 1e-5
_ATTN_EPS = 1e-8


def _ln(x):
    m = jnp.mean(x, axis=-1, keepdims=True)
    v = jnp.mean((x - m) ** 2, axis=-1, keepdims=True)
    return (x - m) * jax.lax.rsqrt(v + _LN_EPS)


def _dot(a, b, dims):
    return jax.lax.dot_general(a, b, (dims, ((), ())),
                               preferred_element_type=jnp.float32)


def _body(slots_ref, x_ref, wq_ref, wk_ref, wv_ref, wu_ref, wg_ref,
          out_ref, attn_ref, *, inv_sqrt_d):
    x = x_ref[0]
    slots = slots_ref[0]

    xin = _ln(x)
    q = _dot(_ln(slots), wq_ref[...], ((1,), (0,)))
    k = _dot(xin, wk_ref[...], ((1,), (0,)))
    v = _dot(xin, wv_ref[...], ((1,), (0,)))

    logits = _dot(q, k, ((1,), (1,))) * inv_sqrt_d
    logits = logits - jnp.max(logits, axis=0, keepdims=True)
    e = jnp.exp(logits)
    attn = e / jnp.sum(e, axis=0, keepdims=True)

    attn_n = attn / (jnp.sum(attn, axis=1, keepdims=True) + _ATTN_EPS)
    updates = _dot(attn_n, v, ((1,), (0,)))
    slots_sa = slots + _dot(updates, wu_ref[...], ((1,), (0,)))

    adj = _dot(attn, attn, ((1,), (1,)))
    adj = adj / (jnp.sum(adj, axis=1, keepdims=True) + _ATTN_EPS)

    agg = _dot(adj, slots_sa, ((1,), (0,)))
    refined = jnp.maximum(_dot(agg, wg_ref[...], ((1,), (0,))), 0.0)

    out_ref[0] = slots_sa + refined
    attn_ref[0] = attn


@jax.jit
def kernel(slots, inputs, Wq, Wk, Wv, Wu, Wg):
    B, K, D = slots.shape
    N = inputs.shape[1] * inputs.shape[2]
    x = inputs.reshape(B, N, D)

    w_spec = pl.BlockSpec((D, D), lambda b: (0, 0))
    out_slots, attn = pl.pallas_call(
        functools.partial(_body, inv_sqrt_d=float(1.0 / (D ** 0.5))),
        grid=(B,),
        in_specs=[
            pl.BlockSpec((1, K, D), lambda b: (b, 0, 0)),
            pl.BlockSpec((1, N, D), lambda b: (b, 0, 0)),
            w_spec, w_spec, w_spec, w_spec, w_spec,
        ],
        out_specs=[
            pl.BlockSpec((1, K, D), lambda b: (b, 0, 0)),
            pl.BlockSpec((1, K, N), lambda b: (b, 0, 0)),
        ],
        out_shape=[
            jax.ShapeDtypeStruct((B, K, D), jnp.float32),
            jax.ShapeDtypeStruct((B, K, N), jnp.float32),
        ],
    )(slots, x, Wq, Wk, Wv, Wu, Wg)
    return out_slots, attn
